# baseline (speedup 1.0000x reference)
import jax
import jax.numpy as jnp
import numpy as np
from jax import lax
from jax.experimental import pallas as pl
from jax.experimental.pallas import tpu as pltpu
from jax.experimental.pallas import tpu_sc as plsc

N_USERS = 25000
N_ITEMS = 25000
N_NODES = N_USERS + N_ITEMS
N_EDGES = 800000
DIM = 64
HALF = DIM // 2
N_LAYERS = 3

NC = 2
NS = 16
PAD_NODES = 50176
ROWS_PER_TILE = PAD_NODES // NS
FLAT = 2 * PAD_NODES

EPT = N_EDGES // NS
CHUNK = 256
STREAM = 128
NSTREAM = CHUNK // STREAM
NCHUNK = -(-EPT // CHUNK)
EPT_PAD = NCHUNK * CHUNK
PAD_E = EPT_PAD - EPT
IDX_ROWS_PER_TILE = EPT_PAD // STREAM
CPB = 14
NBLOCK = NCHUNK // CPB
IRPB = CPB * NSTREAM
EPB = CPB * CHUNK
TRASH_ROW = PAD_NODES - 1
_GDN = lax.GatherDimensionNumbers(
    offset_dims=(), collapsed_slice_dims=(0,), start_index_map=(0,))


def _splat_lane(vec, lane_idx):
    return lax.gather(vec, lane_idx[:, None], _GDN, slice_sizes=(1,),
                      mode=lax.GatherScatterMode.PROMISE_IN_BOUNDS)

MCH = 224
NMCH = ROWS_PER_TILE // MCH
ZFULL = ROWS_PER_TILE // CHUNK
ZREM = ROWS_PER_TILE - ZFULL * CHUNK


def _body(src_hbm, dst_hbm, w_hbm, emb_hbm,
          e1_hbm, e2_hbm, e3_hbm, s_hbm,
          acc, idx_v, dst_v, w_v, rows0, rows1, gsem0, gsem1, ssem0, ssem1):
    c = lax.axis_index("c")
    s = lax.axis_index("s")
    base = c * PAD_NODES
    my_rows = s * ROWS_PER_TILE

    zvec = jnp.zeros((16,), jnp.float32)
    zlane = lax.broadcasted_iota(jnp.int32, (16,), 0) * 0

    def fire_gather(tbl, k, buf, sem):
        pltpu.async_copy(tbl.at[idx_v.at[pl.ds(k * CHUNK, CHUNK)]], buf, sem)

    def drain_chunk(tbl, buf, sem):
        pltpu.make_async_copy(tbl.at[pl.ds(0, CHUNK)], buf, sem).wait()

    def scale_rows(k, buf):
        wo = k * CHUNK

        @plsc.parallel_loop(0, CHUNK // 16, unroll=2)
        def scale(t):
            wvec = w_v[pl.ds(wo + t * 16, 16)]
            for u in range(16):
                e = t * 16 + u
                wv = _splat_lane(wvec, zlane + u)
                buf[e, pl.ds(0, 16)] = buf[e, pl.ds(0, 16)] * wv
                buf[e, pl.ds(16, 16)] = buf[e, pl.ds(16, 16)] * wv

    def fire_scatter(k, buf, sem):
        for j in range(NSTREAM):
            pltpu.async_copy(buf.at[pl.ds(j * STREAM, STREAM)],
                             acc.at[dst_v.at[2 * k + j]], sem, add=True)

    def layer(tbl, out):
        def zero_rows(r, carry):
            rows0[r, pl.ds(0, 16)] = zvec
            rows0[r, pl.ds(16, 16)] = zvec
            rows1[r, pl.ds(0, 16)] = zvec
            rows1[r, pl.ds(16, 16)] = zvec
            return carry

        with jax.named_scope("zero_acc"):
            lax.fori_loop(0, CHUNK, zero_rows, 0)
            for k in range(ZFULL):
                pltpu.sync_copy(rows0 if k % 2 == 0 else rows1,
                                acc.at[pl.ds(my_rows + k * CHUNK, CHUNK)])
            pltpu.sync_copy(rows0.at[pl.ds(0, ZREM)],
                            acc.at[pl.ds(my_rows + ZFULL * CHUNK, ZREM)])
            plsc.subcore_barrier()

        def block_body(b, carry):
            roff = s * IDX_ROWS_PER_TILE + b * IRPB
            eoff = s * EPT_PAD + b * EPB
            pltpu.sync_copy(src_hbm.at[pl.ds(eoff, EPB)], idx_v)
            pltpu.sync_copy(dst_hbm.at[pl.ds(roff, IRPB)], dst_v)
            pltpu.sync_copy(w_hbm.at[pl.ds(eoff, EPB)], w_v)

            def adjust(r, carry2):
                idx_v[pl.ds(r * 16, 16)] = idx_v[pl.ds(r * 16, 16)] + base
                return carry2

            lax.fori_loop(0, EPB // 16, adjust, 0)

            fire_gather(tbl, 0, rows0, gsem0)

            def pair_body(p, carry2):
                c0 = 2 * p

                @pl.when(p > 0)
                def _():
                    drain_chunk(tbl, rows1, ssem1)

                fire_gather(tbl, c0 + 1, rows1, gsem1)
                drain_chunk(tbl, rows0, gsem0)
                scale_rows(c0, rows0)
                fire_scatter(c0, rows0, ssem0)
                drain_chunk(tbl, rows1, gsem1)
                scale_rows(c0 + 1, rows1)
                fire_scatter(c0 + 1, rows1, ssem1)

                @pl.when(p < CPB // 2 - 1)
                def _():
                    drain_chunk(tbl, rows0, ssem0)
                    fire_gather(tbl, c0 + 2, rows0, gsem0)

                return carry2

            lax.fori_loop(0, CPB // 2, pair_body, 0)
            drain_chunk(tbl, rows0, ssem0)
            drain_chunk(tbl, rows1, ssem1)
            return carry

        with jax.named_scope("edges"):
            lax.fori_loop(0, NBLOCK, block_body, 0)
            plsc.subcore_barrier()
        with jax.named_scope("writeback"):
            pltpu.sync_copy(acc.at[pl.ds(my_rows, ROWS_PER_TILE)],
                            out.at[pl.ds(base + my_rows, ROWS_PER_TILE)])
            plsc.subcore_barrier()

    layer(emb_hbm, e1_hbm)
    layer(e1_hbm, e2_hbm)
    layer(e2_hbm, e3_hbm)

    quarter = jnp.float32(0.25)

    def mean_chunk(m, carry):
        roff = base + my_rows + m * MCH
        pltpu.sync_copy(emb_hbm.at[pl.ds(roff, MCH)],
                        rows0.at[pl.ds(0, MCH)])
        for li, tbl in enumerate((e1_hbm, e2_hbm, e3_hbm)):
            pltpu.sync_copy(tbl.at[pl.ds(roff, MCH)],
                            rows1.at[pl.ds(0, MCH)])
            last = li == 2

            def add_rows(r, carry2, _last=last):
                a0 = rows0[r, pl.ds(0, 16)] + rows1[r, pl.ds(0, 16)]
                a1 = rows0[r, pl.ds(16, 16)] + rows1[r, pl.ds(16, 16)]
                if _last:
                    a0 = a0 * quarter
                    a1 = a1 * quarter
                rows0[r, pl.ds(0, 16)] = a0
                rows0[r, pl.ds(16, 16)] = a1
                return carry2

            lax.fori_loop(0, MCH, add_rows, 0)
        pltpu.sync_copy(rows0.at[pl.ds(0, MCH)], s_hbm.at[pl.ds(roff, MCH)])
        return carry

    with jax.named_scope("mean"):
        lax.fori_loop(0, NMCH, mean_chunk, 0)


_f32 = jnp.float32
_gcn = pl.kernel(
    _body,
    out_type=(
        jax.ShapeDtypeStruct((FLAT, HALF), _f32),
        jax.ShapeDtypeStruct((FLAT, HALF), _f32),
        jax.ShapeDtypeStruct((FLAT, HALF), _f32),
        jax.ShapeDtypeStruct((FLAT, HALF), _f32),
    ),
    mesh=plsc.VectorSubcoreMesh(core_axis_name="c", subcore_axis_name="s"),
    compiler_params=pltpu.CompilerParams(use_tc_tiling_on_sc=False),
    scratch_types=[
        pltpu.VMEM_SHARED((PAD_NODES, HALF), _f32),
        pltpu.VMEM((EPB,), jnp.int32),
        pltpu.VMEM((IRPB, STREAM), jnp.int32),
        pltpu.VMEM((EPB,), _f32),
        pltpu.VMEM((CHUNK, HALF), _f32),
        pltpu.VMEM((CHUNK, HALF), _f32),
        pltpu.SemaphoreType.DMA,
        pltpu.SemaphoreType.DMA,
        pltpu.SemaphoreType.DMA,
        pltpu.SemaphoreType.DMA,
    ],
)


@jax.jit
def kernel(edge_index, edge_values, emb_weight):
    src = edge_index[0].astype(jnp.int32).reshape(NS, EPT)
    dst = edge_index[1].astype(jnp.int32).reshape(NS, EPT)
    w = edge_values.reshape(NS, EPT)
    src2d = jnp.pad(src, ((0, 0), (0, PAD_E))).reshape(-1)
    dst2d = jnp.pad(dst, ((0, 0), (0, PAD_E)),
                    constant_values=TRASH_ROW).reshape(-1, STREAM)
    wflat = jnp.pad(w, ((0, 0), (0, PAD_E))).reshape(-1)
    h0 = jnp.pad(emb_weight[:, :HALF], ((0, PAD_NODES - N_NODES), (0, 0)))
    h1 = jnp.pad(emb_weight[:, HALF:], ((0, PAD_NODES - N_NODES), (0, 0)))
    emb_flat = jnp.concatenate([h0, h1], axis=0)

    _e1, _e2, _e3, ssum = _gcn(src2d, dst2d, wflat, emb_flat)
    return jnp.concatenate(
        [ssum[:N_NODES], ssum[PAD_NODES:PAD_NODES + N_NODES]], axis=1)


# speedup vs baseline: 1.0042x; 1.0042x over previous
import jax
import jax.numpy as jnp
import numpy as np
from jax import lax
from jax.experimental import pallas as pl
from jax.experimental.pallas import tpu as pltpu
from jax.experimental.pallas import tpu_sc as plsc

N_USERS = 25000
N_ITEMS = 25000
N_NODES = N_USERS + N_ITEMS
N_EDGES = 800000
DIM = 64
HALF = DIM // 2
N_LAYERS = 3

NC = 2
NS = 16
PAD_NODES = 50176
ROWS_PER_TILE = PAD_NODES // NS
FLAT = 2 * PAD_NODES

EPT = N_EDGES // NS
CHUNK = 256
STREAM = 128
NSTREAM = CHUNK // STREAM
NCHUNK = -(-EPT // CHUNK)
EPT_PAD = NCHUNK * CHUNK
PAD_E = EPT_PAD - EPT
IDX_ROWS_PER_TILE = EPT_PAD // STREAM
CPB = 14
NBLOCK = NCHUNK // CPB
IRPB = CPB * NSTREAM
EPB = CPB * CHUNK
TRASH_ROW = PAD_NODES - 1
_GDN = lax.GatherDimensionNumbers(
    offset_dims=(), collapsed_slice_dims=(0,), start_index_map=(0,))


def _splat_lane(vec, lane_idx):
    return lax.gather(vec, lane_idx[:, None], _GDN, slice_sizes=(1,),
                      mode=lax.GatherScatterMode.PROMISE_IN_BOUNDS)

MCH = 224
NMCH = ROWS_PER_TILE // MCH
ZFULL = ROWS_PER_TILE // CHUNK
ZREM = ROWS_PER_TILE - ZFULL * CHUNK


def _body(src_hbm, dst_hbm, w_hbm, emb_hbm,
          e1_hbm, e2_hbm, e3_hbm, s_hbm,
          acc, idx_v, dst_v, w_v, rows0, rows1, gsem0, gsem1, ssem0, ssem1):
    c = lax.axis_index("c")
    s = lax.axis_index("s")
    base = c * PAD_NODES
    my_rows = s * ROWS_PER_TILE

    zvec = jnp.zeros((16,), jnp.float32)
    zlane = lax.broadcasted_iota(jnp.int32, (16,), 0) * 0

    def fire_gather(tbl, k, buf, sem):
        pltpu.async_copy(tbl.at[idx_v.at[pl.ds(k * CHUNK, CHUNK)]], buf, sem)

    def drain_chunk(tbl, buf, sem):
        pltpu.make_async_copy(tbl.at[pl.ds(0, CHUNK)], buf, sem).wait()

    def scale_rows(k, buf):
        wo = k * CHUNK

        @plsc.parallel_loop(0, CHUNK // 16, unroll=4)
        def scale(t):
            wvec = w_v[pl.ds(wo + t * 16, 16)]
            for u in range(16):
                e = t * 16 + u
                wv = _splat_lane(wvec, zlane + u)
                buf[e, pl.ds(0, 16)] = buf[e, pl.ds(0, 16)] * wv
                buf[e, pl.ds(16, 16)] = buf[e, pl.ds(16, 16)] * wv

    def fire_scatter(k, buf, sem):
        for j in range(NSTREAM):
            pltpu.async_copy(buf.at[pl.ds(j * STREAM, STREAM)],
                             acc.at[dst_v.at[2 * k + j]], sem, add=True)

    def layer(tbl, out):
        def zero_rows(r, carry):
            rows0[r, pl.ds(0, 16)] = zvec
            rows0[r, pl.ds(16, 16)] = zvec
            rows1[r, pl.ds(0, 16)] = zvec
            rows1[r, pl.ds(16, 16)] = zvec
            return carry

        with jax.named_scope("zero_acc"):
            lax.fori_loop(0, CHUNK, zero_rows, 0)
            for k in range(ZFULL):
                pltpu.sync_copy(rows0 if k % 2 == 0 else rows1,
                                acc.at[pl.ds(my_rows + k * CHUNK, CHUNK)])
            pltpu.sync_copy(rows0.at[pl.ds(0, ZREM)],
                            acc.at[pl.ds(my_rows + ZFULL * CHUNK, ZREM)])
            plsc.subcore_barrier()

        def block_body(b, carry):
            roff = s * IDX_ROWS_PER_TILE + b * IRPB
            eoff = s * EPT_PAD + b * EPB
            pltpu.sync_copy(src_hbm.at[pl.ds(eoff, EPB)], idx_v)
            pltpu.sync_copy(dst_hbm.at[pl.ds(roff, IRPB)], dst_v)
            pltpu.sync_copy(w_hbm.at[pl.ds(eoff, EPB)], w_v)

            def adjust(r, carry2):
                idx_v[pl.ds(r * 16, 16)] = idx_v[pl.ds(r * 16, 16)] + base
                return carry2

            lax.fori_loop(0, EPB // 16, adjust, 0)

            fire_gather(tbl, 0, rows0, gsem0)

            def pair_body(p, carry2):
                c0 = 2 * p

                @pl.when(p > 0)
                def _():
                    drain_chunk(tbl, rows1, ssem1)

                fire_gather(tbl, c0 + 1, rows1, gsem1)
                drain_chunk(tbl, rows0, gsem0)
                scale_rows(c0, rows0)
                fire_scatter(c0, rows0, ssem0)
                drain_chunk(tbl, rows1, gsem1)
                scale_rows(c0 + 1, rows1)
                fire_scatter(c0 + 1, rows1, ssem1)

                @pl.when(p < CPB // 2 - 1)
                def _():
                    drain_chunk(tbl, rows0, ssem0)
                    fire_gather(tbl, c0 + 2, rows0, gsem0)

                return carry2

            lax.fori_loop(0, CPB // 2, pair_body, 0)
            drain_chunk(tbl, rows0, ssem0)
            drain_chunk(tbl, rows1, ssem1)
            return carry

        with jax.named_scope("edges"):
            lax.fori_loop(0, NBLOCK, block_body, 0)
            plsc.subcore_barrier()
        with jax.named_scope("writeback"):
            pltpu.sync_copy(acc.at[pl.ds(my_rows, ROWS_PER_TILE)],
                            out.at[pl.ds(base + my_rows, ROWS_PER_TILE)])
            plsc.subcore_barrier()

    layer(emb_hbm, e1_hbm)
    layer(e1_hbm, e2_hbm)
    layer(e2_hbm, e3_hbm)

    quarter = jnp.float32(0.25)

    def mean_chunk(m, carry):
        roff = base + my_rows + m * MCH
        pltpu.sync_copy(emb_hbm.at[pl.ds(roff, MCH)],
                        rows0.at[pl.ds(0, MCH)])
        for li, tbl in enumerate((e1_hbm, e2_hbm, e3_hbm)):
            pltpu.sync_copy(tbl.at[pl.ds(roff, MCH)],
                            rows1.at[pl.ds(0, MCH)])
            last = li == 2

            def add_rows(r, carry2, _last=last):
                a0 = rows0[r, pl.ds(0, 16)] + rows1[r, pl.ds(0, 16)]
                a1 = rows0[r, pl.ds(16, 16)] + rows1[r, pl.ds(16, 16)]
                if _last:
                    a0 = a0 * quarter
                    a1 = a1 * quarter
                rows0[r, pl.ds(0, 16)] = a0
                rows0[r, pl.ds(16, 16)] = a1
                return carry2

            lax.fori_loop(0, MCH, add_rows, 0)
        pltpu.sync_copy(rows0.at[pl.ds(0, MCH)], s_hbm.at[pl.ds(roff, MCH)])
        return carry

    with jax.named_scope("mean"):
        lax.fori_loop(0, NMCH, mean_chunk, 0)


_f32 = jnp.float32
_gcn = pl.kernel(
    _body,
    out_type=(
        jax.ShapeDtypeStruct((FLAT, HALF), _f32),
        jax.ShapeDtypeStruct((FLAT, HALF), _f32),
        jax.ShapeDtypeStruct((FLAT, HALF), _f32),
        jax.ShapeDtypeStruct((FLAT, HALF), _f32),
    ),
    mesh=plsc.VectorSubcoreMesh(core_axis_name="c", subcore_axis_name="s"),
    compiler_params=pltpu.CompilerParams(use_tc_tiling_on_sc=False),
    scratch_types=[
        pltpu.VMEM_SHARED((PAD_NODES, HALF), _f32),
        pltpu.VMEM((EPB,), jnp.int32),
        pltpu.VMEM((IRPB, STREAM), jnp.int32),
        pltpu.VMEM((EPB,), _f32),
        pltpu.VMEM((CHUNK, HALF), _f32),
        pltpu.VMEM((CHUNK, HALF), _f32),
        pltpu.SemaphoreType.DMA,
        pltpu.SemaphoreType.DMA,
        pltpu.SemaphoreType.DMA,
        pltpu.SemaphoreType.DMA,
    ],
)


@jax.jit
def kernel(edge_index, edge_values, emb_weight):
    src = edge_index[0].astype(jnp.int32).reshape(NS, EPT)
    dst = edge_index[1].astype(jnp.int32).reshape(NS, EPT)
    w = edge_values.reshape(NS, EPT)
    src2d = jnp.pad(src, ((0, 0), (0, PAD_E))).reshape(-1)
    dst2d = jnp.pad(dst, ((0, 0), (0, PAD_E)),
                    constant_values=TRASH_ROW).reshape(-1, STREAM)
    wflat = jnp.pad(w, ((0, 0), (0, PAD_E))).reshape(-1)
    h0 = jnp.pad(emb_weight[:, :HALF], ((0, PAD_NODES - N_NODES), (0, 0)))
    h1 = jnp.pad(emb_weight[:, HALF:], ((0, PAD_NODES - N_NODES), (0, 0)))
    emb_flat = jnp.concatenate([h0, h1], axis=0)

    _e1, _e2, _e3, ssum = _gcn(src2d, dst2d, wflat, emb_flat)
    return jnp.concatenate(
        [ssum[:N_NODES], ssum[PAD_NODES:PAD_NODES + N_NODES]], axis=1)


# speedup vs baseline: 1.0381x; 1.0337x over previous
import jax
import jax.numpy as jnp
import numpy as np
from jax import lax
from jax.experimental import pallas as pl
from jax.experimental.pallas import tpu as pltpu
from jax.experimental.pallas import tpu_sc as plsc

N_USERS = 25000
N_ITEMS = 25000
N_NODES = N_USERS + N_ITEMS
N_EDGES = 800000
DIM = 64
HALF = DIM // 2
N_LAYERS = 3

NC = 2
NS = 16
PAD_NODES = 50176
ROWS_PER_TILE = PAD_NODES // NS
FLAT = 2 * PAD_NODES

EPT = N_EDGES // NS
CHUNK = 256
STREAM = 128
NSTREAM = CHUNK // STREAM
NCHUNK = -(-EPT // CHUNK)
EPT_PAD = NCHUNK * CHUNK
PAD_E = EPT_PAD - EPT
IDX_ROWS_PER_TILE = EPT_PAD // STREAM
CPB = 14
NBLOCK = NCHUNK // CPB
IRPB = CPB * NSTREAM
EPB = CPB * CHUNK
TRASH_ROW = PAD_NODES - 1
_GDN = lax.GatherDimensionNumbers(
    offset_dims=(), collapsed_slice_dims=(0,), start_index_map=(0,))


def _splat_lane(vec, lane_idx):
    return lax.gather(vec, lane_idx[:, None], _GDN, slice_sizes=(1,),
                      mode=lax.GatherScatterMode.PROMISE_IN_BOUNDS)

MCH = 224
NMCH = ROWS_PER_TILE // MCH
ZFULL = ROWS_PER_TILE // CHUNK
ZREM = ROWS_PER_TILE - ZFULL * CHUNK


def _body(src_hbm, dst_hbm, w_hbm, emb_hbm,
          e1_hbm, e2_hbm, e3_hbm, s_hbm,
          acc, idx_v, dst_v, w_v, rows0, rows1, gsem0, gsem1, ssem0, ssem1):
    c = lax.axis_index("c")
    s = lax.axis_index("s")
    base = c * PAD_NODES
    my_rows = s * ROWS_PER_TILE

    zvec = jnp.zeros((16,), jnp.float32)
    zlane = lax.broadcasted_iota(jnp.int32, (16,), 0) * 0

    def fire_gather(tbl, k, buf, sem):
        pltpu.async_copy(
            tbl.at[pl.ds(base, PAD_NODES)].at[idx_v.at[pl.ds(k * CHUNK, CHUNK)]],
            buf, sem)

    def drain_chunk(tbl, buf, sem):
        pltpu.make_async_copy(tbl.at[pl.ds(0, CHUNK)], buf, sem).wait()

    def scale_rows(k, buf):
        wo = k * CHUNK

        @plsc.parallel_loop(0, CHUNK // 16, unroll=4)
        def scale(t):
            wvec = w_v[pl.ds(wo + t * 16, 16)]
            for u in range(16):
                e = t * 16 + u
                wv = _splat_lane(wvec, zlane + u)
                buf[e, pl.ds(0, 16)] = buf[e, pl.ds(0, 16)] * wv
                buf[e, pl.ds(16, 16)] = buf[e, pl.ds(16, 16)] * wv

    def fire_scatter(k, buf, sem):
        for j in range(NSTREAM):
            pltpu.async_copy(buf.at[pl.ds(j * STREAM, STREAM)],
                             acc.at[dst_v.at[2 * k + j]], sem, add=True)

    def layer(tbl, out):
        def zero_rows(r, carry):
            rows0[r, pl.ds(0, 16)] = zvec
            rows0[r, pl.ds(16, 16)] = zvec
            rows1[r, pl.ds(0, 16)] = zvec
            rows1[r, pl.ds(16, 16)] = zvec
            return carry

        with jax.named_scope("zero_acc"):
            lax.fori_loop(0, CHUNK, zero_rows, 0)
            for k in range(ZFULL):
                pltpu.sync_copy(rows0 if k % 2 == 0 else rows1,
                                acc.at[pl.ds(my_rows + k * CHUNK, CHUNK)])
            pltpu.sync_copy(rows0.at[pl.ds(0, ZREM)],
                            acc.at[pl.ds(my_rows + ZFULL * CHUNK, ZREM)])
            plsc.subcore_barrier()

        def block_body(b, carry):
            roff = s * IDX_ROWS_PER_TILE + b * IRPB
            eoff = s * EPT_PAD + b * EPB
            pltpu.sync_copy(src_hbm.at[pl.ds(eoff, EPB)], idx_v)
            pltpu.sync_copy(dst_hbm.at[pl.ds(roff, IRPB)], dst_v)
            pltpu.sync_copy(w_hbm.at[pl.ds(eoff, EPB)], w_v)

            fire_gather(tbl, 0, rows0, gsem0)

            def pair_body(p, carry2):
                c0 = 2 * p

                @pl.when(p > 0)
                def _():
                    drain_chunk(tbl, rows1, ssem1)

                fire_gather(tbl, c0 + 1, rows1, gsem1)
                drain_chunk(tbl, rows0, gsem0)
                scale_rows(c0, rows0)
                fire_scatter(c0, rows0, ssem0)
                drain_chunk(tbl, rows1, gsem1)
                scale_rows(c0 + 1, rows1)
                fire_scatter(c0 + 1, rows1, ssem1)

                @pl.when(p < CPB // 2 - 1)
                def _():
                    drain_chunk(tbl, rows0, ssem0)
                    fire_gather(tbl, c0 + 2, rows0, gsem0)

                return carry2

            lax.fori_loop(0, CPB // 2, pair_body, 0)
            drain_chunk(tbl, rows0, ssem0)
            drain_chunk(tbl, rows1, ssem1)
            return carry

        with jax.named_scope("edges"):
            lax.fori_loop(0, NBLOCK, block_body, 0)
            plsc.subcore_barrier()
        with jax.named_scope("writeback"):
            pltpu.sync_copy(acc.at[pl.ds(my_rows, ROWS_PER_TILE)],
                            out.at[pl.ds(base + my_rows, ROWS_PER_TILE)])
            plsc.subcore_barrier()

    layer(emb_hbm, e1_hbm)
    layer(e1_hbm, e2_hbm)
    layer(e2_hbm, e3_hbm)

    quarter = jnp.float32(0.25)

    def mean_chunk(m, carry):
        roff = base + my_rows + m * MCH
        pltpu.sync_copy(emb_hbm.at[pl.ds(roff, MCH)],
                        rows0.at[pl.ds(0, MCH)])
        for li, tbl in enumerate((e1_hbm, e2_hbm, e3_hbm)):
            pltpu.sync_copy(tbl.at[pl.ds(roff, MCH)],
                            rows1.at[pl.ds(0, MCH)])
            last = li == 2

            def add_rows(r, carry2, _last=last):
                a0 = rows0[r, pl.ds(0, 16)] + rows1[r, pl.ds(0, 16)]
                a1 = rows0[r, pl.ds(16, 16)] + rows1[r, pl.ds(16, 16)]
                if _last:
                    a0 = a0 * quarter
                    a1 = a1 * quarter
                rows0[r, pl.ds(0, 16)] = a0
                rows0[r, pl.ds(16, 16)] = a1
                return carry2

            lax.fori_loop(0, MCH, add_rows, 0)
        pltpu.sync_copy(rows0.at[pl.ds(0, MCH)], s_hbm.at[pl.ds(roff, MCH)])
        return carry

    with jax.named_scope("mean"):
        lax.fori_loop(0, NMCH, mean_chunk, 0)


_f32 = jnp.float32
_gcn = pl.kernel(
    _body,
    out_type=(
        jax.ShapeDtypeStruct((FLAT, HALF), _f32),
        jax.ShapeDtypeStruct((FLAT, HALF), _f32),
        jax.ShapeDtypeStruct((FLAT, HALF), _f32),
        jax.ShapeDtypeStruct((FLAT, HALF), _f32),
    ),
    mesh=plsc.VectorSubcoreMesh(core_axis_name="c", subcore_axis_name="s"),
    compiler_params=pltpu.CompilerParams(use_tc_tiling_on_sc=False),
    scratch_types=[
        pltpu.VMEM_SHARED((PAD_NODES, HALF), _f32),
        pltpu.VMEM((EPB,), jnp.int32),
        pltpu.VMEM((IRPB, STREAM), jnp.int32),
        pltpu.VMEM((EPB,), _f32),
        pltpu.VMEM((CHUNK, HALF), _f32),
        pltpu.VMEM((CHUNK, HALF), _f32),
        pltpu.SemaphoreType.DMA,
        pltpu.SemaphoreType.DMA,
        pltpu.SemaphoreType.DMA,
        pltpu.SemaphoreType.DMA,
    ],
)


@jax.jit
def kernel(edge_index, edge_values, emb_weight):
    src = edge_index[0].astype(jnp.int32).reshape(NS, EPT)
    dst = edge_index[1].astype(jnp.int32).reshape(NS, EPT)
    w = edge_values.reshape(NS, EPT)
    src2d = jnp.pad(src, ((0, 0), (0, PAD_E))).reshape(-1)
    dst2d = jnp.pad(dst, ((0, 0), (0, PAD_E)),
                    constant_values=TRASH_ROW).reshape(-1, STREAM)
    wflat = jnp.pad(w, ((0, 0), (0, PAD_E))).reshape(-1)
    h0 = jnp.pad(emb_weight[:, :HALF], ((0, PAD_NODES - N_NODES), (0, 0)))
    h1 = jnp.pad(emb_weight[:, HALF:], ((0, PAD_NODES - N_NODES), (0, 0)))
    emb_flat = jnp.concatenate([h0, h1], axis=0)

    _e1, _e2, _e3, ssum = _gcn(src2d, dst2d, wflat, emb_flat)
    return jnp.concatenate(
        [ssum[:N_NODES], ssum[PAD_NODES:PAD_NODES + N_NODES]], axis=1)
